# baseline (speedup 1.0000x reference)
import functools

import jax
import jax.numpy as jnp
from jax import lax
from jax.experimental import pallas as pl
from jax.experimental.pallas import tpu as pltpu
from jax.experimental.pallas import tpu_sc as plsc

L = 16
NC, NS = 2, 16
NW = NC * NS


@functools.lru_cache(maxsize=None)
def _sc_call(B, I, O, P):
    BW = B // NW
    G = BW // L
    mesh = plsc.VectorSubcoreMesh(core_axis_name="c", subcore_axis_name="s",
                                  num_cores=NC, num_subcores=NS)

    @functools.partial(
        pl.kernel,
        out_type=jax.ShapeDtypeStruct((NW, O * BW), jnp.float32),
        mesh=mesh,
        compiler_params=pltpu.CompilerParams(needs_layout_passes=False),
        scratch_types=[
            pltpu.VMEM((I * BW,), jnp.float32),
            pltpu.VMEM((I * P * O,), jnp.float32),
            pltpu.VMEM((O * BW,), jnp.float32),
            pltpu.VMEM((2 * L,), jnp.float32),
        ],
    )
    def run(xw_hbm, vflat_hbm, params_hbm, out_hbm, x_v, vals_v, out_v, par_v):
        wid = lax.axis_index("s") * NC + lax.axis_index("c")
        pltpu.sync_copy(xw_hbm.at[wid], x_v)
        pltpu.sync_copy(vflat_hbm, vals_v)
        pltpu.sync_copy(params_hbm, par_v)
        p0 = par_v[pl.ds(0, L)]
        inv_dx = par_v[pl.ds(L, L)]
        for g in range(G):
            def body(i, accs, g=g):
                xv = x_v[pl.ds(i * BW + g * L, L)]
                kf = (xv - p0) * inv_dx
                kf = jnp.minimum(jnp.maximum(kf, jnp.float32(0.0)),
                                 jnp.float32(P - 1))
                ki = kf.astype(jnp.int32)
                ki = jnp.minimum(ki, P - 2)
                wv = kf - ki.astype(jnp.float32)
                base = (i * P + ki) * O
                nxt = []
                for o in range(O):
                    y0 = plsc.load_gather(vals_v, [base + o])
                    y1 = plsc.load_gather(vals_v, [base + (O + o)])
                    nxt.append(accs[o] + (y0 + wv * (y1 - y0)))
                return tuple(nxt)

            accs = lax.fori_loop(
                0, I, body,
                tuple(jnp.zeros((L,), jnp.float32) for _ in range(O)))
            for o in range(O):
                out_v[pl.ds(o * BW + g * L, L)] = accs[o]
        pltpu.sync_copy(out_v, out_hbm.at[wid])

    return run


def kernel(x, values, positions):
    B, I = x.shape
    _, O, P = values.shape
    BW = B // NW
    xw = x.reshape(NW, BW, I).transpose(0, 2, 1).reshape(NW, I * BW)
    vflat = values.transpose(0, 2, 1).reshape(I * P * O)
    p0 = positions[0, 0, 0]
    inv_dx = (P - 1) / (positions[0, 0, P - 1] - p0)
    params = jnp.concatenate([
        jnp.full((L,), p0, jnp.float32),
        jnp.full((L,), inv_dx, jnp.float32),
    ])
    out = _sc_call(B, I, O, P)(xw, vflat, params)
    return out.reshape(NW, O, BW).transpose(0, 2, 1).reshape(B, O)


# speedup vs baseline: 1.3052x; 1.3052x over previous
import functools

import jax
import jax.numpy as jnp
from jax import lax
from jax.experimental import pallas as pl
from jax.experimental.pallas import tpu as pltpu
from jax.experimental.pallas import tpu_sc as plsc

L = 16
NC, NS = 2, 16
NW = NC * NS


@functools.lru_cache(maxsize=None)
def _sc_call(B, I, O, P):
    BW = B // NW
    G = BW // L
    mesh = plsc.VectorSubcoreMesh(core_axis_name="c", subcore_axis_name="s",
                                  num_cores=NC, num_subcores=NS)

    @functools.partial(
        pl.kernel,
        out_type=jax.ShapeDtypeStruct((NW, O * BW), jnp.float32),
        mesh=mesh,
        compiler_params=pltpu.CompilerParams(needs_layout_passes=False),
        scratch_types=[
            pltpu.VMEM((I * BW,), jnp.float32),
            pltpu.VMEM((I * P * O,), jnp.float32),
            pltpu.VMEM((O * BW,), jnp.float32),
            pltpu.VMEM((2 * L,), jnp.float32),
        ],
    )
    def run(xw_hbm, vflat_hbm, params_hbm, out_hbm, x_v, vals_v, out_v, par_v):
        wid = lax.axis_index("s") * NC + lax.axis_index("c")
        pltpu.sync_copy(xw_hbm.at[wid], x_v)
        pltpu.sync_copy(vflat_hbm, vals_v)
        pltpu.sync_copy(params_hbm, par_v)
        p0 = par_v[pl.ds(0, L)]
        inv_dx = par_v[pl.ds(L, L)]
        for g in range(G):
            def body(i, accs, g=g):
                xv = x_v[pl.ds(i * BW + g * L, L)]
                kf = (xv - p0) * inv_dx
                kf = jnp.minimum(jnp.maximum(kf, jnp.float32(0.0)),
                                 jnp.float32(P - 1))
                ki = kf.astype(jnp.int32)
                ki = jnp.minimum(ki, P - 2)
                wv = kf - ki.astype(jnp.float32)
                base = i * (O * P) + ki
                nxt = []
                for o in range(O):
                    y0 = plsc.load_gather(vals_v, [base + o * P])
                    y1 = plsc.load_gather(vals_v, [base + (o * P + 1)])
                    nxt.append(accs[o] + (y0 + wv * (y1 - y0)))
                return tuple(nxt)

            accs = lax.fori_loop(
                0, I, body,
                tuple(jnp.zeros((L,), jnp.float32) for _ in range(O)))
            for o in range(O):
                out_v[pl.ds(o * BW + g * L, L)] = accs[o]
        pltpu.sync_copy(out_v, out_hbm.at[wid])

    return run


def kernel(x, values, positions):
    B, I = x.shape
    _, O, P = values.shape
    BW = B // NW
    xw = x.reshape(NW, BW, I).transpose(0, 2, 1).reshape(NW, I * BW)
    vflat = values.reshape(I * P * O)
    p0 = positions[0, 0, 0]
    inv_dx = (P - 1) / (positions[0, 0, P - 1] - p0)
    params = jnp.concatenate([
        jnp.full((L,), p0, jnp.float32),
        jnp.full((L,), inv_dx, jnp.float32),
    ])
    out = _sc_call(B, I, O, P)(xw, vflat, params)
    return out.reshape(NW, O, BW).transpose(0, 2, 1).reshape(B, O)
